# Initial kernel scaffold; baseline (speedup 1.0000x reference)
import jax
import jax.numpy as jnp
from jax.experimental import pallas as pl


def kernel(vectorial_feat, scalar_feat, node_pos, edge_index, W1, b1, W2, b2, Wr, br):
    raise NotImplementedError("write your pallas kernel here")


# speedup vs baseline: 5.8133x; 5.8133x over previous
import functools

import numpy as np
import jax
import jax.numpy as jnp
from jax import lax
from jax.experimental import pallas as pl
from jax.experimental.pallas import tpu as pltpu
from jax.experimental.pallas import tpu_sc as plsc

N = 10000
E = 160000
D = 128
NRB = 20
CUTOFF = 5.0
PI = float(np.pi)

NCHUNK = 4
CF = D // NCHUNK
CW = 3 * CF
NT = 16
NW = 2 * NT

ROWS_PER_TILE = N // NT
EDGES_PER_TILE = E // NT
EB = 80
NBLK = EDGES_PER_TILE // EB

REL_EB = 40
REL_EPT = E // NW
REL_NBLK = REL_EPT // REL_EB

_mesh = plsc.VectorSubcoreMesh(core_axis_name="c", subcore_axis_name="s")


def _z():
    return jnp.int32(0)


def _fori(n, body):
    lax.fori_loop(jnp.int32(0), jnp.int32(n), lambda i, c: (body(i), c)[1],
                  None)


def _rel_pass(posp, src32, tgt32):
    @functools.partial(
        pl.kernel,
        mesh=_mesh,
        compiler_params=pltpu.CompilerParams(use_tc_tiling_on_sc=False),
        out_type=jax.ShapeDtypeStruct((E, 16), jnp.float32),
        scratch_types=[
            pltpu.VMEM((REL_EB,), jnp.int32),
            pltpu.VMEM((REL_EB,), jnp.int32),
            pltpu.VMEM((REL_EB, 16), jnp.float32),
            pltpu.VMEM((REL_EB, 16), jnp.float32),
            pltpu.VMEM((REL_EB, 16), jnp.float32),
            pltpu.SemaphoreType.DMA,
            pltpu.SemaphoreType.DMA,
        ],
    )
    def k(pos_hbm, src_hbm, tgt_hbm, rel_hbm, sbuf, tbuf, abuf, bbuf, rbuf,
          sem1, sem2):
        w = lax.axis_index("s") * jnp.int32(2) + lax.axis_index("c")
        base = w * jnp.int32(REL_EPT)

        def blk(i):
            e0 = base + i * jnp.int32(REL_EB)
            pltpu.sync_copy(src_hbm.at[pl.ds(e0, REL_EB)], sbuf)
            pltpu.sync_copy(tgt_hbm.at[pl.ds(e0, REL_EB)], tbuf)
            c1 = pltpu.async_copy(pos_hbm.at[sbuf], abuf, sem1)
            c2 = pltpu.async_copy(pos_hbm.at[tbuf], bbuf, sem2)
            c1.wait()
            c2.wait()

            def sub(e):
                rbuf[e, :] = bbuf[e, :] - abuf[e, :]

            _fori(REL_EB, sub)
            pltpu.sync_copy(rbuf, rel_hbm.at[pl.ds(e0, REL_EB)])

        _fori(REL_NBLK, blk)

    return k(posp, src32, tgt32)


def _mlp_pass(scalar_feat, W1, b1, W2p, b2p):
    Nb = 2000

    def body(x_ref, w1_ref, b1_ref, w2_ref, b2_ref, o_ref):
        x = x_ref[...]
        t = jnp.dot(x, w1_ref[...], preferred_element_type=jnp.float32)
        t = t + b1_ref[...]
        t = t * jax.nn.sigmoid(t) * (1.0 / 0.6)
        h = jnp.dot(t, w2_ref[...], preferred_element_type=jnp.float32)
        h = h + b2_ref[...]
        for k in range(NCHUNK):
            o_ref[k] = h[:, CW * k:CW * (k + 1)]

    return pl.pallas_call(
        body,
        grid=(N // Nb,),
        in_specs=[
            pl.BlockSpec((Nb, D), lambda i: (i, _z())),
            pl.BlockSpec((D, D), lambda i: (_z(), _z())),
            pl.BlockSpec((1, D), lambda i: (_z(), _z())),
            pl.BlockSpec((D, 3 * D), lambda i: (_z(), _z())),
            pl.BlockSpec((1, 3 * D), lambda i: (_z(), _z())),
        ],
        out_specs=pl.BlockSpec((NCHUNK, Nb, CW), lambda i: (_z(), i, _z())),
        out_shape=jax.ShapeDtypeStruct((NCHUNK, N, CW), jnp.float32),
    )(scalar_feat, W1, b1.reshape(1, D), W2p, b2p.reshape(1, 3 * D))


def _filt_pass(rel16, Wrp, brp):
    Eb = 2000

    def body(rel_ref, wr_ref, br_ref, filt_ref, rn_ref):
        rel = rel_ref[...]
        x = rel[:, 0:1]
        y = rel[:, 1:2]
        z = rel[:, 2:3]
        d2 = x * x + y * y + z * z
        d = jnp.sqrt(d2)
        inv = 1.0 / d
        rn_ref[:, 0:1] = x * inv
        rn_ref[:, 1:2] = y * inv
        rn_ref[:, 2:3] = z * inv
        rn_ref[:, 3:16] = jnp.zeros((x.shape[0], 13), jnp.float32)
        freqs = (lax.broadcasted_iota(jnp.int32, (1, NRB), 1)
                 .astype(jnp.float32) + 1.0) * (PI / CUTOFF)
        rbf = jnp.sin(freqs * d) * inv
        lin = jnp.dot(rbf, wr_ref[...], preferred_element_type=jnp.float32)
        lin = lin + br_ref[...]
        filt = (0.5 * (1.0 + jnp.cos(lin * (PI / CUTOFF)))
                * (lin < CUTOFF).astype(jnp.float32))
        for k in range(NCHUNK):
            filt_ref[k] = filt[:, CW * k:CW * (k + 1)]

    return pl.pallas_call(
        body,
        grid=(E // Eb,),
        in_specs=[
            pl.BlockSpec((Eb, 16), lambda i: (i, _z())),
            pl.BlockSpec((NRB, 3 * D), lambda i: (_z(), _z())),
            pl.BlockSpec((1, 3 * D), lambda i: (_z(), _z())),
        ],
        out_specs=[
            pl.BlockSpec((NCHUNK, Eb, CW), lambda i: (_z(), i, _z())),
            pl.BlockSpec((Eb, 16), lambda i: (i, _z())),
        ],
        out_shape=[
            jax.ShapeDtypeStruct((NCHUNK, E, CW), jnp.float32),
            jax.ShapeDtypeStruct((E, 16), jnp.float32),
        ],
    )(rel16, Wrp, brp.reshape(1, 3 * D))


def _main_pass(p, h4, vf4, filt4, rn4, src32, tgt32):
    @functools.partial(
        pl.kernel,
        mesh=_mesh,
        compiler_params=pltpu.CompilerParams(use_tc_tiling_on_sc=False),
        out_type=jax.ShapeDtypeStruct((2, N, 4 * CF), jnp.float32),
        scratch_types=[
            pltpu.VMEM((EB,), jnp.int32),
            pltpu.VMEM((EB,), jnp.int32),
            pltpu.VMEM((EB,), jnp.int32),
            pltpu.VMEM((EB, CW), jnp.float32),
            pltpu.VMEM((EB, CW), jnp.float32),
            pltpu.VMEM((EB, CW), jnp.float32),
            pltpu.VMEM((EB, 16), jnp.float32),
            pltpu.VMEM((EB, 4 * CF), jnp.float32),
            pltpu.VMEM((125, 4 * CF), jnp.float32),
            pltpu.VMEM_SHARED((N, 4 * CF), jnp.float32),
            pltpu.SemaphoreType.DMA,
            pltpu.SemaphoreType.DMA,
            pltpu.SemaphoreType.DMA,
            pltpu.SemaphoreType.DMA,
        ],
    )
    def k(h_hbm, vf_hbm, filt_hbm, rn_hbm, src_hbm, tgt_hbm, out_hbm,
          sidx, gidx, tidx, hbuf, fbuf, vbuf, rnbuf, obuf, zbuf, acc,
          s1, s2, s3, s4):
        c = lax.axis_index("c")
        s = lax.axis_index("s")
        chunk = c + jnp.int32(2 * p)

        zv = jnp.zeros((16,), jnp.float32)

        def zrow(r):
            for j in range(8):
                zbuf[r, pl.ds(16 * j, 16)] = zv

        _fori(125, zrow)

        def zcopy(i):
            pltpu.sync_copy(zbuf, acc.at[pl.ds(s * jnp.int32(ROWS_PER_TILE) + i * jnp.int32(125), 125)])

        _fori(ROWS_PER_TILE // 125, zcopy)

        plsc.subcore_barrier()

        base = s * jnp.int32(EDGES_PER_TILE)
        off = chunk * jnp.int32(N)

        def blk(i):
            e0 = base + i * jnp.int32(EB)
            pltpu.sync_copy(src_hbm.at[pl.ds(e0, EB)], sidx)
            pltpu.sync_copy(tgt_hbm.at[pl.ds(e0, EB)], tidx)
            for j in range(EB // 16):
                gidx[pl.ds(16 * j, 16)] = sidx[pl.ds(16 * j, 16)] + off
            c1 = pltpu.async_copy(h_hbm.at[gidx], hbuf, s1)
            c2 = pltpu.async_copy(vf_hbm.at[gidx], vbuf, s2)
            c3 = pltpu.async_copy(filt_hbm.at[chunk, pl.ds(e0, EB)], fbuf, s3)
            c4 = pltpu.async_copy(rn_hbm.at[pl.ds(e0, EB)], rnbuf, s4)
            c1.wait()
            c2.wait()
            c3.wait()
            c4.wait()

            def edge(e):
                m = [hbuf[e, pl.ds(16 * j, 16)] * fbuf[e, pl.ds(16 * j, 16)]
                     for j in range(6)]
                obuf[e, pl.ds(0, 16)] = m[0]
                obuf[e, pl.ds(16, 16)] = m[1]
                rnv = rnbuf[e, :]
                for cc in range(3):
                    r = rnv[cc]
                    for hh in range(2):
                        vfv = vbuf[e, pl.ds(CF * cc + 16 * hh, 16)]
                        obuf[e, pl.ds(CF + CF * cc + 16 * hh, 16)] = (
                            m[4 + hh] * r + m[2 + hh] * vfv)

            _fori(EB, edge)
            pltpu.sync_copy(obuf, acc.at[tidx], add=True)

        _fori(NBLK, blk)
        plsc.subcore_barrier()
        r0 = s * jnp.int32(ROWS_PER_TILE)
        pltpu.sync_copy(acc.at[pl.ds(r0, ROWS_PER_TILE)],
                        out_hbm.at[c, pl.ds(r0, ROWS_PER_TILE)])

    return k(h4, vf4, filt4, rn4, src32, tgt32)


def kernel(vectorial_feat, scalar_feat, node_pos, edge_index, W1, b1, W2, b2,
           Wr, br):
    f32 = jnp.float32
    src32 = edge_index[0].astype(jnp.int32)
    tgt32 = edge_index[1].astype(jnp.int32)
    posp = jnp.pad(node_pos.astype(f32), ((0, 0), (0, 13)))

    q = np.arange(3 * D)
    kk = q // CW
    u = (q % CW) // CF
    j = q % CF
    perm = 3 * (CF * kk + j) + u
    W2p = W2[:, perm]
    b2p = b2[perm]
    Wrp = Wr[:, perm]
    brp = br[perm]

    rel16 = _rel_pass(posp, src32, tgt32)
    h4 = _mlp_pass(scalar_feat.astype(f32), W1, b1, W2p, b2p)
    h4 = h4.reshape(NCHUNK * N, CW)
    filt4, rn4 = _filt_pass(rel16, Wrp, brp)
    vf4 = (vectorial_feat.astype(f32)
           .reshape(N, NCHUNK, CF, 3)
           .transpose(1, 0, 3, 2)
           .reshape(NCHUNK * N, CW))

    out0 = _main_pass(0, h4, vf4, filt4, rn4, src32, tgt32)
    out1 = _main_pass(1, h4, vf4, filt4, rn4, src32, tgt32)
    acc = jnp.concatenate([out0, out1], axis=0)

    scalar_message = acc[:, :, :CF].transpose(1, 0, 2).reshape(N, D)
    vectorial_message = (acc[:, :, CF:]
                         .reshape(NCHUNK, N, 3, CF)
                         .transpose(1, 0, 3, 2)
                         .reshape(N, D, 3))
    return (vectorial_message, scalar_message)


# speedup vs baseline: 10.5126x; 1.8084x over previous
import functools

import numpy as np
import jax
import jax.numpy as jnp
from jax import lax
from jax.experimental import pallas as pl
from jax.experimental.pallas import tpu as pltpu
from jax.experimental.pallas import tpu_sc as plsc

N = 10000
E = 160000
D = 128
NRB = 20
CUTOFF = 5.0
PI = float(np.pi)

_COS_C = [0.9999999999993389, -19.739208801726754, 64.9393939719594,
          -85.45681502451104, 60.24459446375221, -26.425691383561002,
          7.899534705902347, -1.6978475376904334, 0.244784399827033]
_SIN_C = [6.283185307177442, -41.341702239859316, 81.6052492362316,
          -76.70585842198088, 42.0586699135306, -15.094388082231236,
          3.8183239132088964, -0.7119140406481951, 0.09117216727193139]

NCHUNK = 4
CF = D // NCHUNK
CW = 3 * CF
NT = 16
NW = 2 * NT

ROWS_PER_TILE = N // NT
EDGES_PER_TILE = E // NT
EB = 40
NBLK = EDGES_PER_TILE // EB

REL_EB = 40
REL_EPT = E // NW
REL_NBLK = REL_EPT // REL_EB

_mesh = plsc.VectorSubcoreMesh(core_axis_name="c", subcore_axis_name="s")


def _z():
    return jnp.int32(0)


def _fori(n, body, unroll=1):
    assert n % unroll == 0

    def step(i, c):
        for u in range(unroll):
            body(i * jnp.int32(unroll) + jnp.int32(u))
        return c

    lax.fori_loop(jnp.int32(0), jnp.int32(n // unroll), step, None)


def _rel_pass(posp, src32, tgt32):
    @functools.partial(
        pl.kernel,
        mesh=_mesh,
        compiler_params=pltpu.CompilerParams(use_tc_tiling_on_sc=False),
        out_type=jax.ShapeDtypeStruct((E, 16), jnp.float32),
        scratch_types=[
            pltpu.VMEM((REL_EB,), jnp.int32),
            pltpu.VMEM((REL_EB,), jnp.int32),
            pltpu.VMEM((REL_EB, 16), jnp.float32),
            pltpu.VMEM((REL_EB, 16), jnp.float32),
            pltpu.VMEM((REL_EB, 16), jnp.float32),
            pltpu.SemaphoreType.DMA,
            pltpu.SemaphoreType.DMA,
        ],
    )
    def k(pos_hbm, src_hbm, tgt_hbm, rel_hbm, sbuf, tbuf, abuf, bbuf, rbuf,
          sem1, sem2):
        w = lax.axis_index("s") * jnp.int32(2) + lax.axis_index("c")
        base = w * jnp.int32(REL_EPT)

        def blk(i):
            e0 = base + i * jnp.int32(REL_EB)
            pltpu.sync_copy(src_hbm.at[pl.ds(e0, REL_EB)], sbuf)
            pltpu.sync_copy(tgt_hbm.at[pl.ds(e0, REL_EB)], tbuf)
            c1 = pltpu.async_copy(pos_hbm.at[sbuf], abuf, sem1)
            c2 = pltpu.async_copy(pos_hbm.at[tbuf], bbuf, sem2)
            c1.wait()
            c2.wait()

            def sub(e):
                rbuf[e, :] = bbuf[e, :] - abuf[e, :]

            _fori(REL_EB, sub)
            pltpu.sync_copy(rbuf, rel_hbm.at[pl.ds(e0, REL_EB)])

        _fori(REL_NBLK, blk)

    return k(posp, src32, tgt32)


def _mlp_pass(scalar_feat, W1, b1, W2p, b2p):
    Nb = 2000

    def body(x_ref, w1_ref, b1_ref, w2_ref, b2_ref, o_ref):
        x = x_ref[...]
        t = jnp.dot(x, w1_ref[...], preferred_element_type=jnp.float32)
        t = t + b1_ref[...]
        t = t * jax.nn.sigmoid(t) * (1.0 / 0.6)
        h = jnp.dot(t, w2_ref[...], preferred_element_type=jnp.float32)
        h = h + b2_ref[...]
        for k in range(NCHUNK):
            o_ref[k] = h[:, CW * k:CW * (k + 1)]

    return pl.pallas_call(
        body,
        grid=(N // Nb,),
        in_specs=[
            pl.BlockSpec((Nb, D), lambda i: (i, _z())),
            pl.BlockSpec((D, D), lambda i: (_z(), _z())),
            pl.BlockSpec((1, D), lambda i: (_z(), _z())),
            pl.BlockSpec((D, 3 * D), lambda i: (_z(), _z())),
            pl.BlockSpec((1, 3 * D), lambda i: (_z(), _z())),
        ],
        out_specs=pl.BlockSpec((NCHUNK, Nb, CW), lambda i: (_z(), i, _z())),
        out_shape=jax.ShapeDtypeStruct((NCHUNK, N, CW), jnp.float32),
    )(scalar_feat, W1, b1.reshape(1, D), W2p, b2p.reshape(1, 3 * D))


def _filt_pass(rel16, Wrp2, brp2, want_rn):
    Eb = 2000

    def body(rel_ref, wr_ref, br_ref, *out_refs):
        rel = rel_ref[...]
        x = rel[:, 0:1]
        y = rel[:, 1:2]
        z = rel[:, 2:3]
        d2 = x * x + y * y + z * z
        d = jnp.sqrt(d2)
        inv = 1.0 / d
        if want_rn:
            rn_ref = out_refs[1]
            rn_ref[:, 0:1] = x * inv
            rn_ref[:, 1:2] = y * inv
            rn_ref[:, 2:3] = z * inv
            rn_ref[:, 3:16] = jnp.zeros((x.shape[0], 13), jnp.float32)
        nover10 = (lax.broadcasted_iota(jnp.int32, (1, NRB), 1)
                   .astype(jnp.float32) + 1.0) * 0.1
        t = d * nover10
        t = t - jnp.round(t)
        s = t * t
        sv = jnp.float32(_SIN_C[-1])
        for coef in _SIN_C[-2::-1]:
            sv = sv * s + jnp.float32(coef)
        rbf = (sv * t) * inv
        lin = jnp.dot(rbf, wr_ref[...], preferred_element_type=jnp.float32)
        lin = lin + br_ref[...]
        tc = lin * jnp.float32(0.1)
        tc = tc - jnp.round(tc)
        sc = tc * tc
        cv = jnp.float32(_COS_C[-1])
        for coef in _COS_C[-2::-1]:
            cv = cv * sc + jnp.float32(coef)
        filt = (0.5 * (1.0 + cv)) * (lin < CUTOFF).astype(jnp.float32)
        filt_ref = out_refs[0]
        for k in range(2):
            filt_ref[k] = filt[:, CW * k:CW * (k + 1)]

    out_specs = [pl.BlockSpec((2, Eb, CW), lambda i: (_z(), i, _z()))]
    out_shape = [jax.ShapeDtypeStruct((2, E, CW), jnp.float32)]
    if want_rn:
        out_specs.append(pl.BlockSpec((Eb, 16), lambda i: (i, _z())))
        out_shape.append(jax.ShapeDtypeStruct((E, 16), jnp.float32))
    return pl.pallas_call(
        body,
        grid=(E // Eb,),
        in_specs=[
            pl.BlockSpec((Eb, 16), lambda i: (i, _z())),
            pl.BlockSpec((NRB, 2 * CW), lambda i: (_z(), _z())),
            pl.BlockSpec((1, 2 * CW), lambda i: (_z(), _z())),
        ],
        out_specs=out_specs,
        out_shape=out_shape,
    )(rel16, Wrp2, brp2.reshape(1, 2 * CW))


def _main_pass(p, h4, vf4, filt2, rn16, srcoff4, tgt32):
    @functools.partial(
        pl.kernel,
        mesh=_mesh,
        compiler_params=pltpu.CompilerParams(use_tc_tiling_on_sc=False),
        out_type=jax.ShapeDtypeStruct((2, N, 4 * CF), jnp.float32),
        scratch_types=[
            pltpu.VMEM((EB,), jnp.int32),
            pltpu.VMEM((EB,), jnp.int32),
            pltpu.VMEM((EB, CW), jnp.float32),
            pltpu.VMEM((EB, CW), jnp.float32),
            pltpu.VMEM((EB, CW), jnp.float32),
            pltpu.VMEM((EB, 16), jnp.float32),
            pltpu.VMEM((EB,), jnp.int32),
            pltpu.VMEM((EB,), jnp.int32),
            pltpu.VMEM((EB, CW), jnp.float32),
            pltpu.VMEM((EB, CW), jnp.float32),
            pltpu.VMEM((EB, CW), jnp.float32),
            pltpu.VMEM((EB, 16), jnp.float32),
            pltpu.VMEM((EB,), jnp.int32),
            pltpu.VMEM((EB, 4 * CF), jnp.float32),
            pltpu.VMEM((25, 4 * CF), jnp.float32),
            pltpu.VMEM_SHARED((N, 4 * CF), jnp.float32),
            pltpu.SemaphoreType.DMA,
            pltpu.SemaphoreType.DMA,
            pltpu.SemaphoreType.DMA,
            pltpu.SemaphoreType.DMA,
        ],
    )
    def k(h_hbm, vf_hbm, filt_hbm, rn_hbm, soff_hbm, tgt_hbm, out_hbm,
          sidxA, tidxA, hbufA, fbufA, vbufA, rnbufA,
          sidxB, tidxB, hbufB, fbufB, vbufB, rnbufB,
          stidx, obuf, zbuf, acc, isemA, gsemA, isemB, gsemB):
        c = lax.axis_index("c")
        s = lax.axis_index("s")
        chunk = c + jnp.int32(2 * p)

        bufA = (sidxA, tidxA, hbufA, fbufA, vbufA, rnbufA, isemA, gsemA)
        bufB = (sidxB, tidxB, hbufB, fbufB, vbufB, rnbufB, isemB, gsemB)

        zv = jnp.zeros((16,), jnp.float32)

        def zrow(r):
            for j in range(8):
                zbuf[r, pl.ds(16 * j, 16)] = zv

        _fori(25, zrow)

        def zcopy(i):
            pltpu.sync_copy(
                zbuf,
                acc.at[pl.ds(s * jnp.int32(ROWS_PER_TILE) + i * jnp.int32(25),
                             25)])

        _fori(ROWS_PER_TILE // 25, zcopy)

        plsc.subcore_barrier()

        base = s * jnp.int32(EDGES_PER_TILE)
        e_last = jnp.int32(E - EB)

        def idx_issue(e0, buf):
            sidx, tidx = buf[0], buf[1]
            isem = buf[6]
            e0 = jnp.minimum(e0, e_last)
            pltpu.async_copy(soff_hbm.at[pl.ds(chunk * jnp.int32(E) + e0, EB)],
                             sidx, isem)
            pltpu.async_copy(tgt_hbm.at[pl.ds(e0, EB)], tidx, isem)

        def idx_wait(buf):
            sidx, tidx = buf[0], buf[1]
            isem = buf[6]
            pltpu.make_async_copy(soff_hbm.at[pl.ds(0, EB)], sidx, isem).wait()
            pltpu.make_async_copy(tgt_hbm.at[pl.ds(0, EB)], tidx, isem).wait()

        def gat_issue(e0, buf):
            sidx, _, hbuf, fbuf, vbuf, rnbuf = buf[:6]
            gsem = buf[7]
            pltpu.async_copy(h_hbm.at[sidx], hbuf, gsem)
            pltpu.async_copy(vf_hbm.at[sidx], vbuf, gsem)
            pltpu.async_copy(filt_hbm.at[c, pl.ds(e0, EB)], fbuf, gsem)
            pltpu.async_copy(rn_hbm.at[pl.ds(e0, EB)], rnbuf, gsem)

        def gat_wait(buf):
            sidx, _, hbuf, fbuf, vbuf, rnbuf = buf[:6]
            gsem = buf[7]
            pltpu.make_async_copy(h_hbm.at[sidx], hbuf, gsem).wait()
            pltpu.make_async_copy(vf_hbm.at[sidx], vbuf, gsem).wait()
            pltpu.make_async_copy(filt_hbm.at[c, pl.ds(0, EB)], fbuf,
                                  gsem).wait()
            pltpu.make_async_copy(rn_hbm.at[pl.ds(0, EB)], rnbuf, gsem).wait()

        _offs = sorted(set(list(range(0, EB - 15, 16)) + [EB - 16]))

        def save_tidx(buf):
            tidx = buf[1]
            for o in _offs:
                stidx[pl.ds(o, 16)] = tidx[pl.ds(o, 16)]

        def compute_scatter(buf):
            _, _, hbuf, fbuf, vbuf, rnbuf = buf[:6]

            def edge(e):
                m = [hbuf[e, pl.ds(16 * j, 16)] * fbuf[e, pl.ds(16 * j, 16)]
                     for j in range(6)]
                obuf[e, pl.ds(0, 16)] = m[0]
                obuf[e, pl.ds(16, 16)] = m[1]
                rnv = rnbuf[e, :]
                for cc in range(3):
                    r = rnv[cc]
                    for hh in range(2):
                        vfv = vbuf[e, pl.ds(CF * cc + 16 * hh, 16)]
                        obuf[e, pl.ds(CF + CF * cc + 16 * hh, 16)] = (
                            m[4 + hh] * r + m[2 + hh] * vfv)

            _fori(EB, edge, unroll=1)
            pltpu.sync_copy(obuf, acc.at[stidx], add=True)

        def e0_of(b):
            return base + b * jnp.int32(EB)

        idx_issue(e0_of(jnp.int32(0)), bufA)
        idx_issue(e0_of(jnp.int32(1)), bufB)
        idx_wait(bufA)
        gat_issue(e0_of(jnp.int32(0)), bufA)

        def pair(j):
            b0 = j * jnp.int32(2)
            idx_wait(bufB)
            gat_issue(e0_of(b0 + jnp.int32(1)), bufB)
            gat_wait(bufA)
            save_tidx(bufA)
            idx_issue(e0_of(b0 + jnp.int32(2)), bufA)
            compute_scatter(bufA)
            idx_wait(bufA)
            gat_issue(e0_of(b0 + jnp.int32(2)), bufA)
            gat_wait(bufB)
            save_tidx(bufB)
            idx_issue(e0_of(b0 + jnp.int32(3)), bufB)
            compute_scatter(bufB)

        _fori((NBLK - 2) // 2, pair)

        gat_wait(bufA)
        save_tidx(bufA)
        compute_scatter(bufA)
        idx_wait(bufB)
        gat_issue(e0_of(jnp.int32(NBLK - 1)), bufB)
        gat_wait(bufB)
        save_tidx(bufB)
        compute_scatter(bufB)

        plsc.subcore_barrier()
        r0 = s * jnp.int32(ROWS_PER_TILE)
        pltpu.sync_copy(acc.at[pl.ds(r0, ROWS_PER_TILE)],
                        out_hbm.at[c, pl.ds(r0, ROWS_PER_TILE)])

    return k(h4, vf4, filt2, rn16, srcoff4.reshape(NCHUNK * E), tgt32)


def kernel(vectorial_feat, scalar_feat, node_pos, edge_index, W1, b1, W2, b2,
           Wr, br):
    f32 = jnp.float32
    src32 = edge_index[0].astype(jnp.int32)
    tgt32 = edge_index[1].astype(jnp.int32)
    posp = jnp.pad(node_pos.astype(f32), ((0, 0), (0, 13)))

    q = np.arange(3 * D)
    kk = q // CW
    u = (q % CW) // CF
    j = q % CF
    perm = 3 * (CF * kk + j) + u
    W2p = W2[:, perm]
    b2p = b2[perm]
    Wrp = Wr[:, perm]
    brp = br[perm]

    rel16 = _rel_pass(posp, src32, tgt32)
    h4 = _mlp_pass(scalar_feat.astype(f32), W1, b1, W2p, b2p)
    h4 = h4.reshape(NCHUNK * N, CW)
    filt01, rn4 = _filt_pass(rel16, Wrp[:, :2 * CW], brp[:2 * CW], True)
    filt23, = _filt_pass(rel16, Wrp[:, 2 * CW:], brp[2 * CW:], False)
    vf4 = (vectorial_feat.astype(f32)
           .reshape(N, NCHUNK, CF, 3)
           .transpose(1, 0, 3, 2)
           .reshape(NCHUNK * N, CW))

    srcoff4 = src32[None, :] + (jnp.arange(NCHUNK, dtype=jnp.int32) * N)[:, None]
    out0 = _main_pass(0, h4, vf4, filt01, rn4, srcoff4, tgt32)
    out1 = _main_pass(1, h4, vf4, filt23, rn4, srcoff4, tgt32)
    acc = jnp.concatenate([out0, out1], axis=0)

    scalar_message = acc[:, :, :CF].transpose(1, 0, 2).reshape(N, D)
    vectorial_message = (acc[:, :, CF:]
                         .reshape(NCHUNK, N, 3, CF)
                         .transpose(1, 0, 3, 2)
                         .reshape(N, D, 3))
    return (vectorial_message, scalar_message)
